# baseline (speedup 1.0000x reference)
import jax
import jax.numpy as jnp
from jax.experimental import pallas as pl

_Q_GRP = 5


def _bcast_add_kernel(qpw_ref, q_ref, out_ref):
    s = q_ref[0] + qpw_ref[0]
    out_ref[...] = jnp.broadcast_to(s[None, :, None, :], out_ref.shape)


def kernel(x, query_pos_weight, queries):
    bs = x.shape[0]
    n_query, embed_dim = query_pos_weight.shape
    q_blk = n_query // _Q_GRP
    qpw = query_pos_weight.reshape(_Q_GRP, q_blk, embed_dim)
    q = queries.reshape(_Q_GRP, q_blk, embed_dim)
    out = pl.pallas_call(
        _bcast_add_kernel,
        grid=(_Q_GRP,),
        in_specs=[
            pl.BlockSpec((1, q_blk, embed_dim), lambda i: (i, 0, 0)),
            pl.BlockSpec((1, q_blk, embed_dim), lambda i: (i, 0, 0)),
        ],
        out_specs=pl.BlockSpec((1, q_blk, bs, embed_dim), lambda i: (i, 0, 0, 0)),
        out_shape=jax.ShapeDtypeStruct(
            (_Q_GRP, q_blk, bs, embed_dim), queries.dtype
        ),
    )(qpw, q)
    return jnp.swapaxes(out.reshape(n_query, bs, embed_dim), 0, 1)
